# baseline (speedup 1.0000x reference)
import functools

import jax
import jax.numpy as jnp
from jax import lax
from jax.experimental import pallas as pl
from jax.experimental.pallas import tpu as pltpu
from jax.experimental.pallas import tpu_sc as plsc

T = 2048
D = 1024
E = 8
TOPK = 2
DFF = 512
NG = 4
TG = 2
NSH = 2
RSF = 2.5

TB = 256
NBLK = T * TOPK // TB + E
NP = NBLK * TB

NC = 2
NS = 16
NW = NC * NS
APW = T * TOPK // NW
TPW = T // NW


def _ranks(vals):
    n = vals.shape[-1]
    a = vals[..., :, None]
    b = vals[..., None, :]
    idx = lax.broadcasted_iota(jnp.int32, (n, n), 0)
    jdx = lax.broadcasted_iota(jnp.int32, (n, n), 1)
    beats = (b > a) | ((b == a) & (jdx < idx))
    return jnp.sum(beats.astype(jnp.int32), axis=-1)


def _gate_body(x_ref, gw_ref, bias_ref, pos_ref, wts_ref, bexp_ref, bval_ref):
    x = x_ref[...]
    logits = lax.dot_general(x, gw_ref[...], (((1,), (1,)), ((), ())),
                             preferred_element_type=jnp.float32)
    scores = jax.nn.sigmoid(logits)
    sfc = scores + bias_ref[...]
    grow = lax.broadcasted_iota(jnp.int32, (NG, E), 0)
    gcol = lax.broadcasted_iota(jnp.int32, (NG, E), 1)
    M = (gcol // (E // NG) == grow).astype(jnp.float32)
    gm = lax.dot_general(sfc, M, (((1,), (1,)), ((), ())),
                         preferred_element_type=jnp.float32,
                         precision=lax.Precision.HIGHEST)
    gsel = _ranks(gm) < TG
    emaskf = lax.dot_general(gsel.astype(jnp.float32), M,
                             (((1,), (0,)), ((), ())),
                             preferred_element_type=jnp.float32)
    masked = jnp.where(emaskf > 0.5, sfc, -jnp.inf)
    rk = _ranks(masked)
    sel0 = (rk == 0).astype(jnp.float32)
    sel1 = (rk == 1).astype(jnp.float32)
    w0 = jnp.sum(scores * sel0, axis=1, keepdims=True)
    w1 = jnp.sum(scores * sel1, axis=1, keepdims=True)
    denom = w0[:, 0] + w1[:, 0]
    denom = (denom + 1e-20).reshape(w0.shape)
    wcat = jnp.concatenate([w0 / denom * RSF, w1 / denom * RSF],
                           axis=0)
    wts_ref[...] = jnp.broadcast_to(wcat, (TOPK * T, 16))

    eself = (rk < TOPK).astype(jnp.float32)
    cnt = jnp.sum(eself, axis=0, keepdims=True)
    padded = jnp.floor((cnt + (TB - 1)) / TB) * TB
    li = lax.broadcasted_iota(jnp.int32, (E, E), 0)
    lj = lax.broadcasted_iota(jnp.int32, (E, E), 1)
    Ltri = (li < lj).astype(jnp.float32)
    off = lax.dot_general(padded, Ltri, (((1,), (0,)), ((), ())),
                          preferred_element_type=jnp.float32,
                          precision=lax.Precision.HIGHEST)
    cb = 256
    ci = lax.broadcasted_iota(jnp.int32, (cb, cb), 0)
    cj = lax.broadcasted_iota(jnp.int32, (cb, cb), 1)
    Lincl = (ci >= cj).astype(jnp.float32)
    carry = jnp.zeros((1, E), jnp.float32)
    csum_parts = []
    for i in range(T // cb):
        blk = eself[i * cb:(i + 1) * cb, :]
        csum_parts.append(
            lax.dot_general(Lincl, blk, (((1,), (0,)), ((), ())),
                            preferred_element_type=jnp.float32) + carry)
        carry = carry + jnp.sum(blk, axis=0, keepdims=True)
    csum = jnp.concatenate(csum_parts, axis=0)
    rank = csum - eself
    posmat = rank + off
    pos0 = jnp.sum(posmat * sel0, axis=1, keepdims=True)
    pos1 = jnp.sum(posmat * sel1, axis=1, keepdims=True)
    pos_ref[...] = jnp.concatenate([pos0, pos1], axis=0).astype(jnp.int32)

    ends = off + padded
    rends = off + cnt
    bstart = (lax.broadcasted_iota(jnp.int32, (NBLK, 1), 0) * TB
              ).astype(jnp.float32)
    bexp = jnp.sum((bstart >= ends).astype(jnp.int32), axis=1, keepdims=True)
    bexp = jnp.minimum(bexp, E - 1)
    ecol = lax.broadcasted_iota(jnp.int32, (NBLK, E), 1)
    bvalid = jnp.sum(((ecol == bexp) & (bstart < rends)).astype(jnp.int32),
                     axis=1, keepdims=True)
    bexp_ref[...] = bexp
    bval_ref[...] = bvalid


def _dispatch_body(x_hbm, pos_hbm, xg_hbm, pos_v, xs_v, sem):
    wid = lax.axis_index("s") * NC + lax.axis_index("c")
    tb0 = (wid * APW) % T
    pltpu.sync_copy(pos_hbm.at[pl.ds(wid * 2, 2), :], pos_v)
    for j in range(2):
        pltpu.sync_copy(x_hbm.at[pl.ds(tb0 + j * 64, 64), :], xs_v)
        pltpu.async_copy(xs_v, xg_hbm.at[pos_v.at[j]], sem).wait()


def _combine_body(eg_hbm, pos_hbm, w_hbm, sh_hbm, out_hbm,
                  p_v, w0x_v, w1x_v, g0_v, g1_v, sh_v, o_v, sem):
    wid = lax.axis_index("s") * NC + lax.axis_index("c")
    tb0 = wid * TPW
    pltpu.sync_copy(pos_hbm.at[wid], p_v.at[0])
    pltpu.sync_copy(pos_hbm.at[NW + wid], p_v.at[1])
    for cg in range(4):
        pltpu.async_copy(eg_hbm.at[p_v.at[0, pl.ds(cg * 16, 16)]],
                         g0_v, sem).wait()
        pltpu.async_copy(eg_hbm.at[p_v.at[1, pl.ds(cg * 16, 16)]],
                         g1_v, sem).wait()
        pltpu.sync_copy(sh_hbm.at[pl.ds(tb0 + cg * 16, 16), :], sh_v)
        pltpu.sync_copy(w_hbm.at[pl.ds(tb0 + cg * 16, 16), :], w0x_v)
        pltpu.sync_copy(w_hbm.at[pl.ds(T + tb0 + cg * 16, 16), :], w1x_v)

        def body(i, _):
            w0 = w0x_v[i, :]
            w1 = w1x_v[i, :]
            for dc in range(D // 16):
                sl = pl.ds(dc * 16, 16)
                o_v[i, sl] = g0_v[i, sl] * w0 + g1_v[i, sl] * w1 + sh_v[i, sl]
            return 0

        lax.fori_loop(0, 16, body, 0)
        pltpu.sync_copy(o_v, out_hbm.at[pl.ds(tb0 + cg * 16, 16), :])


def _grouped_body(bexp_s, bval_s, xg_ref, w13_ref, w2_ref, eg_ref):
    b = pl.program_id(0)

    @pl.when(bval_s[b] > 0)
    def _():
        xg = xg_ref[...]
        gu = lax.dot_general(xg, w13_ref[0], (((1,), (1,)), ((), ())),
                             preferred_element_type=jnp.float32)
        g = gu[:, :DFF]
        u = gu[:, DFF:]
        act = g * jax.nn.sigmoid(g) * u
        eg_ref[...] = lax.dot_general(act.astype(jnp.bfloat16), w2_ref[0],
                                      (((1,), (1,)), ((), ())),
                                      preferred_element_type=jnp.float32)


def _shared_body(x_ref, sgu_ref, sdn_ref, out_ref):
    x = x_ref[...]
    gu = lax.dot_general(x, sgu_ref[...], (((1,), (1,)), ((), ())),
                         preferred_element_type=jnp.float32)
    h = DFF * NSH
    g = gu[:, :h]
    u = gu[:, h:]
    act = g * jax.nn.sigmoid(g) * u
    out_ref[...] = lax.dot_general(act.astype(jnp.bfloat16), sdn_ref[...],
                                   (((1,), (1,)), ((), ())),
                                   preferred_element_type=jnp.float32)


@functools.cache
def _sc_kernels():
    mesh = plsc.VectorSubcoreMesh(core_axis_name="c", subcore_axis_name="s",
                                  num_cores=NC, num_subcores=NS)
    dispatch = pl.kernel(
        _dispatch_body,
        out_type=jax.ShapeDtypeStruct((NP, D), jnp.float32),
        mesh=mesh,
        scratch_types=[
            pltpu.VMEM((2, APW // 2), jnp.int32),
            pltpu.VMEM((APW // 2, D), jnp.float32),
            pltpu.SemaphoreType.DMA,
        ],
    )
    combine = pl.kernel(
        _combine_body,
        out_type=jax.ShapeDtypeStruct((T, D), jnp.float32),
        mesh=mesh,
        scratch_types=[
            pltpu.VMEM((2, TPW), jnp.int32),
            pltpu.VMEM((16, 16), jnp.float32),
            pltpu.VMEM((16, 16), jnp.float32),
            pltpu.VMEM((16, D), jnp.float32),
            pltpu.VMEM((16, D), jnp.float32),
            pltpu.VMEM((16, D), jnp.float32),
            pltpu.VMEM((16, D), jnp.float32),
            pltpu.SemaphoreType.DMA,
        ],
    )
    return dispatch, combine


@jax.jit
def kernel(hidden_states, gate_weight, e_score_correction_bias, w13, w2,
           shared_gate_up, shared_down):
    x = hidden_states
    bias2d = e_score_correction_bias.reshape(1, E)

    pos, wts, bexp, bvalid = pl.pallas_call(
        _gate_body,
        out_shape=(
            jax.ShapeDtypeStruct((TOPK * T, 1), jnp.int32),
            jax.ShapeDtypeStruct((TOPK * T, 16), jnp.float32),
            jax.ShapeDtypeStruct((NBLK, 1), jnp.int32),
            jax.ShapeDtypeStruct((NBLK, 1), jnp.int32),
        ),
    )(x, gate_weight, bias2d)

    pos64 = pos.reshape(TOPK * T // 64, 64)
    bexp1 = bexp.reshape(NBLK)
    bval1 = bvalid.reshape(NBLK)

    _dispatch, _combine = _sc_kernels()
    xg = _dispatch(x, pos64)

    eg = pl.pallas_call(
        _grouped_body,
        grid_spec=pltpu.PrefetchScalarGridSpec(
            num_scalar_prefetch=2,
            grid=(NBLK,),
            in_specs=[
                pl.BlockSpec((TB, D), lambda b, be, bv: (b, 0)),
                pl.BlockSpec((1, 2 * DFF, D), lambda b, be, bv: (be[b], 0, 0)),
                pl.BlockSpec((1, D, DFF), lambda b, be, bv: (be[b], 0, 0)),
            ],
            out_specs=pl.BlockSpec((TB, D), lambda b, be, bv: (b, 0)),
        ),
        out_shape=jax.ShapeDtypeStruct((NP, D), jnp.float32),
    )(bexp1, bval1, xg, w13, w2)

    tb2 = 512
    sh = pl.pallas_call(
        _shared_body,
        grid=(T // tb2,),
        in_specs=[
            pl.BlockSpec((tb2, D), lambda t: (t, 0)),
            pl.BlockSpec((2 * DFF * NSH, D), lambda t: (0, 0)),
            pl.BlockSpec((D, DFF * NSH), lambda t: (0, 0)),
        ],
        out_specs=pl.BlockSpec((tb2, D), lambda t: (t, 0)),
        out_shape=jax.ShapeDtypeStruct((T, D), jnp.float32),
    )(x, shared_gate_up, shared_down)

    return _combine(eg, pos64, wts, sh)


# speedup vs baseline: 1.1095x; 1.1095x over previous
import functools

import jax
import jax.numpy as jnp
from jax import lax
from jax.experimental import pallas as pl
from jax.experimental.pallas import tpu as pltpu
from jax.experimental.pallas import tpu_sc as plsc

T = 2048
D = 1024
E = 8
TOPK = 2
DFF = 512
NG = 4
TG = 2
NSH = 2
RSF = 2.5

TB = 256
NBLK = T * TOPK // TB + E
NP = NBLK * TB

NC = 2
NS = 16
NW = NC * NS
APW = T * TOPK // NW
TPW = T // NW


def _ranks(vals):
    n = vals.shape[-1]
    a = vals[..., :, None]
    b = vals[..., None, :]
    idx = lax.broadcasted_iota(jnp.int32, (n, n), 0)
    jdx = lax.broadcasted_iota(jnp.int32, (n, n), 1)
    beats = (b > a) | ((b == a) & (jdx < idx))
    return jnp.sum(beats.astype(jnp.int32), axis=-1)


def _gate_body(x_ref, gw_ref, bias_ref, pos_ref, w0_ref, w1_ref,
               bexp_ref, bval_ref):
    x = x_ref[...]
    logits = lax.dot_general(x, gw_ref[...], (((1,), (1,)), ((), ())),
                             preferred_element_type=jnp.float32)
    scores = jax.nn.sigmoid(logits)
    sfc = scores + bias_ref[...]
    grow = lax.broadcasted_iota(jnp.int32, (NG, E), 0)
    gcol = lax.broadcasted_iota(jnp.int32, (NG, E), 1)
    M = (gcol // (E // NG) == grow).astype(jnp.float32)
    gm = lax.dot_general(sfc, M, (((1,), (1,)), ((), ())),
                         preferred_element_type=jnp.float32,
                         precision=lax.Precision.HIGHEST)
    gsel = _ranks(gm) < TG
    emaskf = lax.dot_general(gsel.astype(jnp.float32), M,
                             (((1,), (0,)), ((), ())),
                             preferred_element_type=jnp.float32)
    masked = jnp.where(emaskf > 0.5, sfc, -jnp.inf)
    rk = _ranks(masked)
    sel0 = (rk == 0).astype(jnp.float32)
    sel1 = (rk == 1).astype(jnp.float32)
    w0 = jnp.sum(scores * sel0, axis=1, keepdims=True)
    w1 = jnp.sum(scores * sel1, axis=1, keepdims=True)
    denom = w0[:, 0] + w1[:, 0]
    denom = (denom + 1e-20).reshape(w0.shape)
    w0_ref[...] = w0 / denom * RSF
    w1_ref[...] = w1 / denom * RSF

    eself = (rk < TOPK).astype(jnp.float32)
    cnt = jnp.sum(eself, axis=0, keepdims=True)
    padded = jnp.floor((cnt + (TB - 1)) / TB) * TB
    li = lax.broadcasted_iota(jnp.int32, (E, E), 0)
    lj = lax.broadcasted_iota(jnp.int32, (E, E), 1)
    Ltri = (li < lj).astype(jnp.float32)
    off = lax.dot_general(padded, Ltri, (((1,), (0,)), ((), ())),
                          preferred_element_type=jnp.float32,
                          precision=lax.Precision.HIGHEST)
    cb = 256
    ci = lax.broadcasted_iota(jnp.int32, (cb, cb), 0)
    cj = lax.broadcasted_iota(jnp.int32, (cb, cb), 1)
    Lincl = (ci >= cj).astype(jnp.float32)
    carry = jnp.zeros((1, E), jnp.float32)
    csum_parts = []
    for i in range(T // cb):
        blk = eself[i * cb:(i + 1) * cb, :]
        csum_parts.append(
            lax.dot_general(Lincl, blk, (((1,), (0,)), ((), ())),
                            preferred_element_type=jnp.float32) + carry)
        carry = carry + jnp.sum(blk, axis=0, keepdims=True)
    csum = jnp.concatenate(csum_parts, axis=0)
    rank = csum - eself
    posmat = rank + off
    pos0 = jnp.sum(posmat * sel0, axis=1, keepdims=True)
    pos1 = jnp.sum(posmat * sel1, axis=1, keepdims=True)
    pos_ref[...] = jnp.concatenate([pos0, pos1], axis=0).astype(jnp.int32)

    ends = off + padded
    rends = off + cnt
    bstart = (lax.broadcasted_iota(jnp.int32, (NBLK, 1), 0) * TB
              ).astype(jnp.float32)
    bexp = jnp.sum((bstart >= ends).astype(jnp.int32), axis=1, keepdims=True)
    bexp = jnp.minimum(bexp, E - 1)
    ecol = lax.broadcasted_iota(jnp.int32, (NBLK, E), 1)
    bvalid = jnp.sum(((ecol == bexp) & (bstart < rends)).astype(jnp.int32),
                     axis=1, keepdims=True)
    bexp_ref[...] = bexp
    bval_ref[...] = bvalid


def _dispatch_body(x_hbm, pos_hbm, xg_hbm, pos_v, xs_v, sem):
    wid = lax.axis_index("s") * NC + lax.axis_index("c")
    tb0 = (wid * APW) % T
    pltpu.sync_copy(pos_hbm.at[pl.ds(wid * 2, 2), :], pos_v)
    for j in range(2):
        pltpu.sync_copy(x_hbm.at[pl.ds(tb0 + j * 64, 64), :], xs_v)
        pltpu.async_copy(xs_v, xg_hbm.at[pos_v.at[j]], sem).wait()


def _backmap_body(eg_hbm, pos_hbm, r0_hbm, r1_hbm,
                  p_v, g0_v, g1_v, sem):
    wid = lax.axis_index("s") * NC + lax.axis_index("c")
    tb0 = wid * TPW
    pltpu.sync_copy(pos_hbm.at[wid], p_v.at[0])
    pltpu.sync_copy(pos_hbm.at[NW + wid], p_v.at[1])
    for cg in range(2):
        cp0 = eg_hbm.at[p_v.at[0, pl.ds(cg * 32, 32)]]
        cp1 = eg_hbm.at[p_v.at[1, pl.ds(cg * 32, 32)]]
        c0 = pltpu.async_copy(cp0, g0_v, sem)
        c1 = pltpu.async_copy(cp1, g1_v, sem)
        c0.wait()
        c1.wait()
        pltpu.sync_copy(g0_v, r0_hbm.at[pl.ds(tb0 + cg * 32, 32), :])
        pltpu.sync_copy(g1_v, r1_hbm.at[pl.ds(tb0 + cg * 32, 32), :])


def _grouped_body(bexp_s, bval_s, xg_ref, w13_ref, w2_ref, eg_ref):
    b = pl.program_id(0)

    @pl.when(bval_s[b] > 0)
    def _():
        xg = xg_ref[...]
        gu = lax.dot_general(xg, w13_ref[0], (((1,), (1,)), ((), ())),
                             preferred_element_type=jnp.float32)
        g = gu[:, :DFF]
        u = gu[:, DFF:]
        act = g * jax.nn.sigmoid(g) * u
        eg_ref[...] = lax.dot_general(act.astype(jnp.bfloat16), w2_ref[0],
                                      (((1,), (1,)), ((), ())),
                                      preferred_element_type=jnp.float32)


def _shared_body(x_ref, sgu_ref, sdn_ref, r0_ref, r1_ref, w0_ref, w1_ref,
                 out_ref):
    x = x_ref[...]
    gu = lax.dot_general(x, sgu_ref[...], (((1,), (1,)), ((), ())),
                         preferred_element_type=jnp.float32)
    h = DFF * NSH
    g = gu[:, :h]
    u = gu[:, h:]
    act = g * jax.nn.sigmoid(g) * u
    sh = lax.dot_general(act.astype(jnp.bfloat16), sdn_ref[...],
                         (((1,), (1,)), ((), ())),
                         preferred_element_type=jnp.float32)
    out_ref[...] = (sh + r0_ref[...] * w0_ref[...]
                    + r1_ref[...] * w1_ref[...])


@functools.cache
def _sc_kernels():
    mesh = plsc.VectorSubcoreMesh(core_axis_name="c", subcore_axis_name="s",
                                  num_cores=NC, num_subcores=NS)
    dispatch = pl.kernel(
        _dispatch_body,
        out_type=jax.ShapeDtypeStruct((NP, D), jnp.float32),
        mesh=mesh,
        scratch_types=[
            pltpu.VMEM((2, APW // 2), jnp.int32),
            pltpu.VMEM((APW // 2, D), jnp.float32),
            pltpu.SemaphoreType.DMA,
        ],
    )
    backmap = pl.kernel(
        _backmap_body,
        out_type=(jax.ShapeDtypeStruct((T, D), jnp.float32),
                  jax.ShapeDtypeStruct((T, D), jnp.float32)),
        mesh=mesh,
        scratch_types=[
            pltpu.VMEM((2, TPW), jnp.int32),
            pltpu.VMEM((32, D), jnp.float32),
            pltpu.VMEM((32, D), jnp.float32),
            pltpu.SemaphoreType.DMA,
        ],
    )
    return dispatch, backmap


@jax.jit
def kernel(hidden_states, gate_weight, e_score_correction_bias, w13, w2,
           shared_gate_up, shared_down):
    x = hidden_states
    bias2d = e_score_correction_bias.reshape(1, E)

    pos, w0c, w1c, bexp, bvalid = pl.pallas_call(
        _gate_body,
        out_shape=(
            jax.ShapeDtypeStruct((TOPK * T, 1), jnp.int32),
            jax.ShapeDtypeStruct((T, 1), jnp.float32),
            jax.ShapeDtypeStruct((T, 1), jnp.float32),
            jax.ShapeDtypeStruct((NBLK, 1), jnp.int32),
            jax.ShapeDtypeStruct((NBLK, 1), jnp.int32),
        ),
    )(x, gate_weight, bias2d)

    pos64 = pos.reshape(TOPK * T // 64, 64)
    bexp1 = bexp.reshape(NBLK)
    bval1 = bvalid.reshape(NBLK)

    _dispatch, _backmap = _sc_kernels()
    xg = _dispatch(x, pos64)

    eg = pl.pallas_call(
        _grouped_body,
        grid_spec=pltpu.PrefetchScalarGridSpec(
            num_scalar_prefetch=2,
            grid=(NBLK,),
            in_specs=[
                pl.BlockSpec((TB, D), lambda b, be, bv: (b, 0)),
                pl.BlockSpec((1, 2 * DFF, D), lambda b, be, bv: (be[b], 0, 0)),
                pl.BlockSpec((1, D, DFF), lambda b, be, bv: (be[b], 0, 0)),
            ],
            out_specs=pl.BlockSpec((TB, D), lambda b, be, bv: (b, 0)),
        ),
        out_shape=jax.ShapeDtypeStruct((NP, D), jnp.float32),
    )(bexp1, bval1, xg, w13, w2)

    r0, r1 = _backmap(eg, pos64)

    tb2 = 512
    out = pl.pallas_call(
        _shared_body,
        grid=(T // tb2,),
        in_specs=[
            pl.BlockSpec((tb2, D), lambda t: (t, 0)),
            pl.BlockSpec((2 * DFF * NSH, D), lambda t: (0, 0)),
            pl.BlockSpec((D, DFF * NSH), lambda t: (0, 0)),
            pl.BlockSpec((tb2, D), lambda t: (t, 0)),
            pl.BlockSpec((tb2, D), lambda t: (t, 0)),
            pl.BlockSpec((tb2, 1), lambda t: (t, 0)),
            pl.BlockSpec((tb2, 1), lambda t: (t, 0)),
        ],
        out_specs=pl.BlockSpec((tb2, D), lambda t: (t, 0)),
        out_shape=jax.ShapeDtypeStruct((T, D), jnp.float32),
    )(x, shared_gate_up, shared_down, r0, r1, w0c, w1c)
    return out


# speedup vs baseline: 1.2228x; 1.1022x over previous
import functools

import jax
import jax.numpy as jnp
from jax import lax
from jax.experimental import pallas as pl
from jax.experimental.pallas import tpu as pltpu
from jax.experimental.pallas import tpu_sc as plsc

T = 2048
D = 1024
E = 8
TOPK = 2
DFF = 512
NG = 4
TG = 2
NSH = 2
RSF = 2.5

TB = 256
NBLK = T * TOPK // TB + E
NP = NBLK * TB

NC = 2
NS = 16
NW = NC * NS
APW = T * TOPK // NW
TPW = T // NW


def _ranks(vals):
    t, n = vals.shape
    cols = n * n
    b_t = jnp.concatenate([vals] * n, axis=1)
    a_r = jnp.concatenate(
        [jnp.broadcast_to(vals[:, i:i + 1], (t, n)) for i in range(n)],
        axis=1)
    lane = lax.broadcasted_iota(jnp.int32, (t, cols), 1)
    tie_lower = (lane % n) < (lane // n)
    beats = (b_t > a_r) | ((b_t == a_r) & tie_lower)
    mrow = lax.broadcasted_iota(jnp.int32, (cols, n), 0)
    mcol = lax.broadcasted_iota(jnp.int32, (cols, n), 1)
    M2 = (mrow // n == mcol).astype(jnp.float32)
    rank = lax.dot_general(beats.astype(jnp.float32), M2,
                           (((1,), (0,)), ((), ())),
                           preferred_element_type=jnp.float32)
    return rank.astype(jnp.int32)


def _gate_body(x_ref, gw_ref, bias_ref, pos_ref, w0_ref, w1_ref,
               bexp_ref, bval_ref):
    x = x_ref[...]
    logits = lax.dot_general(x, gw_ref[...], (((1,), (1,)), ((), ())),
                             preferred_element_type=jnp.float32)
    scores = jax.nn.sigmoid(logits)
    sfc = scores + bias_ref[...]
    grow = lax.broadcasted_iota(jnp.int32, (NG, E), 0)
    gcol = lax.broadcasted_iota(jnp.int32, (NG, E), 1)
    M = (gcol // (E // NG) == grow).astype(jnp.float32)
    gm = lax.dot_general(sfc, M, (((1,), (1,)), ((), ())),
                         preferred_element_type=jnp.float32,
                         precision=lax.Precision.HIGHEST)
    gsel = _ranks(gm) < TG
    emaskf = lax.dot_general(gsel.astype(jnp.float32), M,
                             (((1,), (0,)), ((), ())),
                             preferred_element_type=jnp.float32)
    masked = jnp.where(emaskf > 0.5, sfc, -jnp.inf)
    rk = _ranks(masked)
    sel0 = (rk == 0).astype(jnp.float32)
    sel1 = (rk == 1).astype(jnp.float32)
    w0 = jnp.sum(scores * sel0, axis=1, keepdims=True)
    w1 = jnp.sum(scores * sel1, axis=1, keepdims=True)
    denom = w0[:, 0] + w1[:, 0]
    denom = (denom + 1e-20).reshape(w0.shape)
    w0_ref[...] = w0 / denom * RSF
    w1_ref[...] = w1 / denom * RSF

    eself = (rk < TOPK).astype(jnp.float32)
    cnt = jnp.sum(eself, axis=0, keepdims=True)
    padded = jnp.floor((cnt + (TB - 1)) / TB) * TB
    li = lax.broadcasted_iota(jnp.int32, (E, E), 0)
    lj = lax.broadcasted_iota(jnp.int32, (E, E), 1)
    Ltri = (li < lj).astype(jnp.float32)
    off = lax.dot_general(padded, Ltri, (((1,), (0,)), ((), ())),
                          preferred_element_type=jnp.float32,
                          precision=lax.Precision.HIGHEST)
    cb = 512
    ci = lax.broadcasted_iota(jnp.int32, (cb, cb), 0)
    cj = lax.broadcasted_iota(jnp.int32, (cb, cb), 1)
    Lincl = (ci >= cj).astype(jnp.float32)
    carry = jnp.zeros((1, E), jnp.float32)
    csum_parts = []
    for i in range(T // cb):
        blk = eself[i * cb:(i + 1) * cb, :]
        csum_parts.append(
            lax.dot_general(Lincl, blk, (((1,), (0,)), ((), ())),
                            preferred_element_type=jnp.float32) + carry)
        carry = carry + jnp.sum(blk, axis=0, keepdims=True)
    csum = jnp.concatenate(csum_parts, axis=0)
    rank = csum - eself
    posmat = rank + off
    pos0 = jnp.sum(posmat * sel0, axis=1, keepdims=True)
    pos1 = jnp.sum(posmat * sel1, axis=1, keepdims=True)
    pos_ref[...] = jnp.concatenate([pos0, pos1], axis=0).astype(jnp.int32)

    ends = off + padded
    rends = off + cnt
    bstart = (lax.broadcasted_iota(jnp.int32, (NBLK, 1), 0) * TB
              ).astype(jnp.float32)
    bexp = jnp.sum((bstart >= ends).astype(jnp.int32), axis=1, keepdims=True)
    bexp = jnp.minimum(bexp, E - 1)
    ecol = lax.broadcasted_iota(jnp.int32, (NBLK, E), 1)
    bvalid = jnp.sum(((ecol == bexp) & (bstart < rends)).astype(jnp.int32),
                     axis=1, keepdims=True)
    bexp_ref[...] = bexp
    bval_ref[...] = bvalid


def _dispatch_body(x_hbm, pos_hbm, xg_hbm, pos_v, xs_v, sem):
    wid = lax.axis_index("s") * NC + lax.axis_index("c")
    tb0 = (wid * APW) % T
    pltpu.sync_copy(pos_hbm.at[pl.ds(wid * 2, 2), :], pos_v)
    for j in range(2):
        pltpu.sync_copy(x_hbm.at[pl.ds(tb0 + j * 64, 64), :], xs_v)
        pltpu.async_copy(xs_v, xg_hbm.at[pos_v.at[j]], sem).wait()


def _backmap_body(eg_hbm, pos_hbm, r0_hbm, r1_hbm,
                  p_v, g0_v, g1_v, sem):
    wid = lax.axis_index("s") * NC + lax.axis_index("c")
    tb0 = wid * TPW
    pltpu.sync_copy(pos_hbm.at[wid], p_v.at[0])
    pltpu.sync_copy(pos_hbm.at[NW + wid], p_v.at[1])
    for cg in range(2):
        cp0 = eg_hbm.at[p_v.at[0, pl.ds(cg * 32, 32)]]
        cp1 = eg_hbm.at[p_v.at[1, pl.ds(cg * 32, 32)]]
        c0 = pltpu.async_copy(cp0, g0_v, sem)
        c1 = pltpu.async_copy(cp1, g1_v, sem)
        c0.wait()
        c1.wait()
        pltpu.sync_copy(g0_v, r0_hbm.at[pl.ds(tb0 + cg * 32, 32), :])
        pltpu.sync_copy(g1_v, r1_hbm.at[pl.ds(tb0 + cg * 32, 32), :])


def _grouped_body(bexp_s, bval_s, xg_ref, w13_ref, w2_ref, eg_ref):
    b = pl.program_id(0)

    @pl.when(bval_s[b] > 0)
    def _():
        xg = xg_ref[...]
        gu = lax.dot_general(xg, w13_ref[0], (((1,), (1,)), ((), ())),
                             preferred_element_type=jnp.float32)
        g = gu[:, :DFF]
        u = gu[:, DFF:]
        act = g * jax.nn.sigmoid(g) * u
        eg_ref[...] = lax.dot_general(act.astype(jnp.bfloat16), w2_ref[0],
                                      (((1,), (1,)), ((), ())),
                                      preferred_element_type=jnp.float32)


def _shared_body(x_ref, sgu_ref, sdn_ref, out_ref):
    x = x_ref[...]
    gu = lax.dot_general(x, sgu_ref[...], (((1,), (1,)), ((), ())),
                         preferred_element_type=jnp.float32)
    h = DFF * NSH
    g = gu[:, :h]
    u = gu[:, h:]
    act = g * jax.nn.sigmoid(g) * u
    out_ref[...] = lax.dot_general(act.astype(jnp.bfloat16), sdn_ref[...],
                                   (((1,), (1,)), ((), ())),
                                   preferred_element_type=jnp.float32)


def _add_body(sh_ref, r0_ref, r1_ref, w0_ref, w1_ref, out_ref):
    out_ref[...] = (sh_ref[...] + r0_ref[...] * w0_ref[...]
                    + r1_ref[...] * w1_ref[...])


@functools.cache
def _sc_kernels():
    mesh = plsc.VectorSubcoreMesh(core_axis_name="c", subcore_axis_name="s",
                                  num_cores=NC, num_subcores=NS)
    dispatch = pl.kernel(
        _dispatch_body,
        out_type=jax.ShapeDtypeStruct((NP, D), jnp.float32),
        mesh=mesh,
        scratch_types=[
            pltpu.VMEM((2, APW // 2), jnp.int32),
            pltpu.VMEM((APW // 2, D), jnp.float32),
            pltpu.SemaphoreType.DMA,
        ],
    )
    backmap = pl.kernel(
        _backmap_body,
        out_type=(jax.ShapeDtypeStruct((T, D), jnp.float32),
                  jax.ShapeDtypeStruct((T, D), jnp.float32)),
        mesh=mesh,
        scratch_types=[
            pltpu.VMEM((2, TPW), jnp.int32),
            pltpu.VMEM((32, D), jnp.float32),
            pltpu.VMEM((32, D), jnp.float32),
            pltpu.SemaphoreType.DMA,
        ],
    )
    return dispatch, backmap


@jax.jit
def kernel(hidden_states, gate_weight, e_score_correction_bias, w13, w2,
           shared_gate_up, shared_down):
    x = hidden_states
    bias2d = e_score_correction_bias.reshape(1, E)

    pos, w0c, w1c, bexp, bvalid = pl.pallas_call(
        _gate_body,
        out_shape=(
            jax.ShapeDtypeStruct((TOPK * T, 1), jnp.int32),
            jax.ShapeDtypeStruct((T, 1), jnp.float32),
            jax.ShapeDtypeStruct((T, 1), jnp.float32),
            jax.ShapeDtypeStruct((NBLK, 1), jnp.int32),
            jax.ShapeDtypeStruct((NBLK, 1), jnp.int32),
        ),
    )(x, gate_weight, bias2d)

    pos64 = pos.reshape(TOPK * T // 64, 64)
    bexp1 = bexp.reshape(NBLK)
    bval1 = bvalid.reshape(NBLK)

    tb2 = 512
    sh = pl.pallas_call(
        _shared_body,
        grid=(T // tb2,),
        in_specs=[
            pl.BlockSpec((tb2, D), lambda t: (t, 0)),
            pl.BlockSpec((2 * DFF * NSH, D), lambda t: (0, 0)),
            pl.BlockSpec((D, DFF * NSH), lambda t: (0, 0)),
        ],
        out_specs=pl.BlockSpec((tb2, D), lambda t: (t, 0)),
        out_shape=jax.ShapeDtypeStruct((T, D), jnp.float32),
    )(x, shared_gate_up, shared_down)

    _dispatch, _backmap = _sc_kernels()
    xg = _dispatch(x, pos64)

    eg = pl.pallas_call(
        _grouped_body,
        grid_spec=pltpu.PrefetchScalarGridSpec(
            num_scalar_prefetch=2,
            grid=(NBLK,),
            in_specs=[
                pl.BlockSpec((TB, D), lambda b, be, bv: (b, 0)),
                pl.BlockSpec((1, 2 * DFF, D), lambda b, be, bv: (be[b], 0, 0)),
                pl.BlockSpec((1, D, DFF), lambda b, be, bv: (be[b], 0, 0)),
            ],
            out_specs=pl.BlockSpec((TB, D), lambda b, be, bv: (b, 0)),
        ),
        out_shape=jax.ShapeDtypeStruct((NP, D), jnp.float32),
    )(bexp1, bval1, xg, w13, w2)

    r0, r1 = _backmap(eg, pos64)

    out = pl.pallas_call(
        _add_body,
        grid=(T // tb2,),
        in_specs=[
            pl.BlockSpec((tb2, D), lambda t: (t, 0)),
            pl.BlockSpec((tb2, D), lambda t: (t, 0)),
            pl.BlockSpec((tb2, D), lambda t: (t, 0)),
            pl.BlockSpec((tb2, 1), lambda t: (t, 0)),
            pl.BlockSpec((tb2, 1), lambda t: (t, 0)),
        ],
        out_specs=pl.BlockSpec((tb2, D), lambda t: (t, 0)),
        out_shape=jax.ShapeDtypeStruct((T, D), jnp.float32),
    )(sh, r0, r1, w0c, w1c)
    return out
